# baseline (speedup 1.0000x reference)
import functools

import jax
import jax.numpy as jnp
from jax import lax
from jax.experimental import pallas as pl
from jax.experimental.pallas import tpu as pltpu
from jax.experimental.pallas import tpu_sc as plsc

_N = 20000
_E = 40000
_DH = 64
_DP = 128
_NPAD = 20480
_EPAD = 40960
_TILE = 512
_SUB = 128
_WCHUNK = 1280
_NSUB = _WCHUNK // _SUB
_GROW = 640
_QTR = _NPAD // 4
_SCHUNK = _EPAD // 16
_SROWS = _SCHUNK // _SUB
_ACC = 5248
_ZSTR = _ACC // 16


def _gather_body(nodes_hbm, idx_hbm, out_hbm, idx_v, rows_v, sem):
    wid = lax.axis_index("s") * 2 + lax.axis_index("c")
    pltpu.sync_copy(idx_hbm.at[wid], idx_v)
    for g in range(_WCHUNK // _GROW):
        descs = [
            pltpu.async_copy(
                nodes_hbm.at[idx_v.at[g * 5 + j]],
                rows_v.at[pl.ds(j * _SUB, _SUB)],
                sem,
            )
            for j in range(5)
        ]
        for d in descs:
            d.wait()
        pltpu.sync_copy(
            rows_v, out_hbm.at[pl.ds(wid * _WCHUNK + g * _GROW, _GROW)])


def _scatter_body(msg_hbm, idx_hbm, zeros_hbm, out_hbm, idx_v, rows_v, acc_sh):
    c = lax.axis_index("c")
    s = lax.axis_index("s")
    stripe = _QTR // 16
    for p in range(2):
        pltpu.sync_copy(idx_hbm.at[p * 32 + c * 16 + s], idx_v)
        pltpu.sync_copy(zeros_hbm, acc_sh.at[pl.ds(s * _ZSTR, _ZSTR)])
        plsc.subcore_barrier()
        for g in range(_SCHUNK // _GROW):
            pltpu.sync_copy(
                msg_hbm.at[pl.ds(s * _SCHUNK + g * _GROW, _GROW)], rows_v)
            for j in range(5):
                pltpu.sync_copy(
                    rows_v.at[pl.ds(j * _SUB, _SUB)],
                    acc_sh.at[idx_v.at[g * 5 + j]],
                    add=True,
                )
        plsc.subcore_barrier()
        pltpu.sync_copy(
            acc_sh.at[pl.ds(s * stripe, stripe)],
            out_hbm.at[p * 2 + c, pl.ds(s * stripe, stripe)],
        )
        plsc.subcore_barrier()


@functools.lru_cache(maxsize=None)
def _sc_kernels():
    mesh = plsc.VectorSubcoreMesh(core_axis_name="c", subcore_axis_name="s")
    gather = pl.kernel(
        _gather_body,
        out_type=jax.ShapeDtypeStruct((_EPAD, _DP), jnp.float32),
        mesh=mesh,
        scratch_types=[
            pltpu.VMEM((16, _SUB), jnp.int32),
            pltpu.VMEM((_GROW, _DP), jnp.float32),
            pltpu.SemaphoreType.DMA,
        ],
    )
    scatter = pl.kernel(
        _scatter_body,
        out_type=jax.ShapeDtypeStruct((4, _QTR, _DP), jnp.float32),
        mesh=mesh,
        scratch_types=[
            pltpu.VMEM((_SROWS, _SUB), jnp.int32),
            pltpu.VMEM((_GROW, _DP), jnp.float32),
            pltpu.VMEM_SHARED((_ACC, _DP), jnp.float32),
        ],
    )
    return gather, scatter


def _dense_relu_body(x_ref, w_ref, b_ref, o_ref, *, pad):
    y = jnp.dot(x_ref[...], w_ref[...], preferred_element_type=jnp.float32)
    y = jnp.maximum(y + b_ref[...], 0.0)
    if pad:
        y = jnp.concatenate(
            [y, jnp.zeros((y.shape[0], _DP - _DH), jnp.float32)], axis=1)
    o_ref[...] = y


def _dense_relu(x, w, b, pad):
    rows, din = x.shape
    dout = _DP if pad else _DH
    return pl.pallas_call(
        functools.partial(_dense_relu_body, pad=pad),
        grid=(rows // _TILE,),
        in_specs=[
            pl.BlockSpec((_TILE, din), lambda i: (i, 0)),
            pl.BlockSpec((din, _DH), lambda i: (0, 0)),
            pl.BlockSpec((1, _DH), lambda i: (0, 0)),
        ],
        out_specs=pl.BlockSpec((_TILE, dout), lambda i: (i, 0)),
        out_shape=jax.ShapeDtypeStruct((rows, dout), jnp.float32),
    )(x, w, b.reshape(1, _DH))


def _msg_body(x_ref, h_ref, t2_ref, bm_ref, o_ref):
    x = x_ref[:, :_DH]
    h = h_ref[...]
    h_rep = jnp.concatenate(
        [jnp.broadcast_to(h[:, k:k + 1], (_TILE, _DH)) for k in range(_DH)],
        axis=1,
    )
    x_til = jnp.concatenate([x] * _DH, axis=1)
    msg = jnp.dot(h_rep * x_til, t2_ref[...],
                  preferred_element_type=jnp.float32)
    msg = msg + jnp.dot(x, bm_ref[...], preferred_element_type=jnp.float32)
    eid = pl.program_id(0) * _TILE + lax.broadcasted_iota(
        jnp.int32, (_TILE, 1), 0)
    msg = jnp.where(eid < _E, msg, 0.0)
    o_ref[...] = jnp.concatenate(
        [msg, jnp.zeros((_TILE, _DP - _DH), jnp.float32)], axis=1)


def _msg_tc(xe, h, t2, bm):
    return pl.pallas_call(
        _msg_body,
        grid=(_EPAD // _TILE,),
        in_specs=[
            pl.BlockSpec((_TILE, _DP), lambda i: (i, 0)),
            pl.BlockSpec((_TILE, _DH), lambda i: (i, 0)),
            pl.BlockSpec((_DH * _DH, _DH), lambda i: (0, 0)),
            pl.BlockSpec((_DH, _DH), lambda i: (0, 0)),
        ],
        out_specs=pl.BlockSpec((_TILE, _DP), lambda i: (i, 0)),
        out_shape=jax.ShapeDtypeStruct((_EPAD, _DP), jnp.float32),
    )(xe, h, t2, bm)


def _update_body(agg_ref, out_ref, wr_ref, cb_ref, wt_ref, wb_ref, bm_ref,
                 init_ref, o_ref, *, last):
    agg = agg_ref[:, :_DH]
    out = out_ref[:, :_DH]
    conv = agg + jnp.dot(out, wr_ref[...],
                         preferred_element_type=jnp.float32) + cb_ref[...]
    m = jnp.maximum(conv, 0.0)
    new = (jnp.dot(m, wt_ref[...], preferred_element_type=jnp.float32)
           + jnp.dot(out, wb_ref[...], preferred_element_type=jnp.float32)
           + bm_ref[...])
    if last:
        new = new + init_ref[...]
    o_ref[...] = jnp.concatenate(
        [new, jnp.zeros((_TILE, _DP - _DH), jnp.float32)], axis=1)


def _update_tc(agg, out, wr, cb, wt, wb, bm, init, last):
    full = lambda i: (0, 0)
    return pl.pallas_call(
        functools.partial(_update_body, last=last),
        grid=(_NPAD // _TILE,),
        in_specs=[
            pl.BlockSpec((_TILE, _DP), lambda i: (i, 0)),
            pl.BlockSpec((_TILE, _DP), lambda i: (i, 0)),
            pl.BlockSpec((_DH, _DH), full),
            pl.BlockSpec((1, _DH), full),
            pl.BlockSpec((_DH, _DH), full),
            pl.BlockSpec((_DH, _DH), full),
            pl.BlockSpec((1, _DH), full),
            pl.BlockSpec((_TILE, _DH), lambda i: (i, 0)),
        ],
        out_specs=pl.BlockSpec((_TILE, _DP), lambda i: (i, 0)),
        out_shape=jax.ShapeDtypeStruct((_NPAD, _DP), jnp.float32),
    )(agg, out, wr, cb.reshape(1, _DH), wt, wb, bm.reshape(1, _DH), init)


def kernel(node_features, edge_attr, edge_index, W_in, b_in, W_msg, b_msg,
           W_e1, b_e1, W_e2, b_e2, W_root, conv_bias):
    f32 = jnp.float32
    x_pad = jnp.pad(node_features, ((0, _NPAD - _N), (0, 0)))
    ea_pad = jnp.pad(edge_attr, ((0, _EPAD - _E), (0, 0)))

    srcf = jnp.pad(edge_index[0], (0, _EPAD - _E)).reshape(32, _NSUB, _SUB)
    src = jnp.pad(srcf, ((0, 0), (0, 16 - _NSUB), (0, 0)))

    dstf = jnp.pad(edge_index[1], (0, _EPAD - _E))
    bases = jnp.array([[0], [_QTR], [2 * _QTR], [3 * _QTR]], jnp.int32)
    rel = dstf[None, :] - bases
    routed = jnp.where((rel >= 0) & (rel < _QTR), rel, _QTR)
    dst = routed.astype(jnp.int32).reshape(2, 2, 16, _SROWS, _SUB)
    dst = dst.reshape(64, _SROWS, _SUB)

    t2 = W_e2.reshape(_DH * _DH, _DH)
    bm = b_e2.reshape(_DH, _DH)
    wt = W_msg[:_DH]
    wb = W_msg[_DH:]
    zeros_stripe = jnp.zeros((_ZSTR, _DP), f32)

    sc_gather, sc_scatter = _sc_kernels()
    out = _dense_relu(x_pad, W_in, b_in, pad=True)
    h = _dense_relu(ea_pad, W_e1, b_e1, pad=False)
    for step in range(3):
        xe = sc_gather(out, src)
        msg = _msg_tc(xe, h, t2, bm)
        parts = sc_scatter(msg, dst, zeros_stripe)
        agg = parts.reshape(_NPAD, _DP)
        out = _update_tc(agg, out, W_root, conv_bias, wt, wb, b_msg,
                         x_pad, last=(step == 2))
    return out[:_N, :_DH]


# speedup vs baseline: 1.1094x; 1.1094x over previous
import functools

import jax
import jax.numpy as jnp
from jax import lax
from jax.experimental import pallas as pl
from jax.experimental.pallas import tpu as pltpu
from jax.experimental.pallas import tpu_sc as plsc

_N = 20000
_E = 40000
_DH = 64
_DP = 128
_NPAD = 20480
_EPAD = 40960
_TILE = 512
_SUB = 128
_WCHUNK = 1280
_NSUB = _WCHUNK // _SUB
_GROW = 640
_QTR = _NPAD // 4
_SCHUNK = _EPAD // 16
_SROWS = _SCHUNK // _SUB
_ACC = 5248
_ZSTR = _ACC // 16


def _gather_body(nodes_hbm, idx_hbm, out_hbm, idx_v, rows_v, sem):
    wid = lax.axis_index("s") * 2 + lax.axis_index("c")
    pltpu.sync_copy(idx_hbm.at[wid], idx_v)
    for g in range(_WCHUNK // _GROW):
        descs = [
            pltpu.async_copy(
                nodes_hbm.at[idx_v.at[g * 5 + j]],
                rows_v.at[pl.ds(j * _SUB, _SUB)],
                sem,
            )
            for j in range(5)
        ]
        for d in descs:
            d.wait()
        pltpu.sync_copy(
            rows_v, out_hbm.at[pl.ds(wid * _WCHUNK + g * _GROW, _GROW)])


def _scatter_body(msg_hbm, idx_hbm, zeros_hbm, out_hbm, idx_v, rows_v, acc_sh):
    c = lax.axis_index("c")
    s = lax.axis_index("s")
    stripe = _QTR // 16
    for p in range(2):
        pltpu.sync_copy(idx_hbm.at[p * 32 + c * 16 + s], idx_v)
        pltpu.sync_copy(zeros_hbm, acc_sh.at[pl.ds(s * _ZSTR, _ZSTR)])
        plsc.subcore_barrier()
        for g in range(_SCHUNK // _GROW):
            pltpu.sync_copy(
                msg_hbm.at[pl.ds(s * _SCHUNK + g * _GROW, _GROW)], rows_v)
            for j in range(5):
                pltpu.sync_copy(
                    rows_v.at[pl.ds(j * _SUB, _SUB)],
                    acc_sh.at[idx_v.at[g * 5 + j]],
                    add=True,
                )
        plsc.subcore_barrier()
        pltpu.sync_copy(
            acc_sh.at[pl.ds(s * stripe, stripe)],
            out_hbm.at[p * 2 + c, pl.ds(s * stripe, stripe)],
        )
        plsc.subcore_barrier()


@functools.lru_cache(maxsize=None)
def _sc_kernels():
    mesh = plsc.VectorSubcoreMesh(core_axis_name="c", subcore_axis_name="s")
    gather = pl.kernel(
        _gather_body,
        out_type=jax.ShapeDtypeStruct((_EPAD, _DP), jnp.float32),
        mesh=mesh,
        scratch_types=[
            pltpu.VMEM((16, _SUB), jnp.int32),
            pltpu.VMEM((_GROW, _DP), jnp.float32),
            pltpu.SemaphoreType.DMA,
        ],
    )
    scatter = pl.kernel(
        _scatter_body,
        out_type=jax.ShapeDtypeStruct((4, _QTR, _DP), jnp.float32),
        mesh=mesh,
        scratch_types=[
            pltpu.VMEM((_SROWS, _SUB), jnp.int32),
            pltpu.VMEM((_GROW, _DP), jnp.float32),
            pltpu.VMEM_SHARED((_ACC, _DP), jnp.float32),
        ],
    )
    return gather, scatter


def _dense_relu_body(x_ref, w_ref, b_ref, o_ref, *, pad):
    y = jnp.dot(x_ref[...], w_ref[...], preferred_element_type=jnp.float32)
    y = jnp.maximum(y + b_ref[...], 0.0)
    y = y.astype(o_ref.dtype)
    if pad:
        y = jnp.concatenate(
            [y, jnp.zeros((y.shape[0], _DP - _DH), o_ref.dtype)], axis=1)
    o_ref[...] = y


def _dense_relu(x, w, b, pad, out_dtype=jnp.float32):
    rows, din = x.shape
    dout = _DP if pad else _DH
    return pl.pallas_call(
        functools.partial(_dense_relu_body, pad=pad),
        grid=(rows // _TILE,),
        in_specs=[
            pl.BlockSpec((_TILE, din), lambda i: (i, 0)),
            pl.BlockSpec((din, _DH), lambda i: (0, 0)),
            pl.BlockSpec((1, _DH), lambda i: (0, 0)),
        ],
        out_specs=pl.BlockSpec((_TILE, dout), lambda i: (i, 0)),
        out_shape=jax.ShapeDtypeStruct((rows, dout), out_dtype),
    )(x, w, b.reshape(1, _DH))


def _msg_body(x_ref, h_ref, t2_ref, bm_ref, r_ref, s_ref, o_ref):
    x = x_ref[:, :_DH]
    xb = x.astype(jnp.bfloat16)
    hb = h_ref[...]
    h_rep = jnp.dot(hb, r_ref[...], preferred_element_type=jnp.float32)
    x_til = jnp.dot(xb, s_ref[...], preferred_element_type=jnp.float32)
    pmat = (h_rep * x_til).astype(jnp.bfloat16)
    msg = jnp.dot(pmat, t2_ref[...], preferred_element_type=jnp.float32)
    msg = msg + jnp.dot(x, bm_ref[...], preferred_element_type=jnp.float32)
    eid = pl.program_id(0) * _TILE + lax.broadcasted_iota(
        jnp.int32, (_TILE, 1), 0)
    msg = jnp.where(eid < _E, msg, 0.0)
    o_ref[...] = jnp.concatenate(
        [msg, jnp.zeros((_TILE, _DP - _DH), jnp.float32)], axis=1)


def _msg_tc(xe, h, t2, bm, rmat, smat):
    return pl.pallas_call(
        _msg_body,
        grid=(_EPAD // _TILE,),
        in_specs=[
            pl.BlockSpec((_TILE, _DP), lambda i: (i, 0)),
            pl.BlockSpec((_TILE, _DH), lambda i: (i, 0)),
            pl.BlockSpec((_DH * _DH, _DH), lambda i: (0, 0)),
            pl.BlockSpec((_DH, _DH), lambda i: (0, 0)),
            pl.BlockSpec((_DH, _DH * _DH), lambda i: (0, 0)),
            pl.BlockSpec((_DH, _DH * _DH), lambda i: (0, 0)),
        ],
        out_specs=pl.BlockSpec((_TILE, _DP), lambda i: (i, 0)),
        out_shape=jax.ShapeDtypeStruct((_EPAD, _DP), jnp.float32),
    )(xe, h, t2, bm, rmat, smat)


def _update_body(agg_ref, out_ref, wr_ref, cb_ref, wt_ref, wb_ref, bm_ref,
                 init_ref, o_ref, *, last):
    agg = agg_ref[:, :_DH]
    out = out_ref[:, :_DH]
    conv = agg + jnp.dot(out, wr_ref[...],
                         preferred_element_type=jnp.float32) + cb_ref[...]
    m = jnp.maximum(conv, 0.0)
    new = (jnp.dot(m, wt_ref[...], preferred_element_type=jnp.float32)
           + jnp.dot(out, wb_ref[...], preferred_element_type=jnp.float32)
           + bm_ref[...])
    if last:
        new = new + init_ref[...]
    o_ref[...] = jnp.concatenate(
        [new, jnp.zeros((_TILE, _DP - _DH), jnp.float32)], axis=1)


def _update_tc(agg, out, wr, cb, wt, wb, bm, init, last):
    full = lambda i: (0, 0)
    return pl.pallas_call(
        functools.partial(_update_body, last=last),
        grid=(_NPAD // _TILE,),
        in_specs=[
            pl.BlockSpec((_TILE, _DP), lambda i: (i, 0)),
            pl.BlockSpec((_TILE, _DP), lambda i: (i, 0)),
            pl.BlockSpec((_DH, _DH), full),
            pl.BlockSpec((1, _DH), full),
            pl.BlockSpec((_DH, _DH), full),
            pl.BlockSpec((_DH, _DH), full),
            pl.BlockSpec((1, _DH), full),
            pl.BlockSpec((_TILE, _DH), lambda i: (i, 0)),
        ],
        out_specs=pl.BlockSpec((_TILE, _DP), lambda i: (i, 0)),
        out_shape=jax.ShapeDtypeStruct((_NPAD, _DP), jnp.float32),
    )(agg, out, wr, cb.reshape(1, _DH), wt, wb, bm.reshape(1, _DH), init)


def kernel(node_features, edge_attr, edge_index, W_in, b_in, W_msg, b_msg,
           W_e1, b_e1, W_e2, b_e2, W_root, conv_bias):
    f32 = jnp.float32
    x_pad = jnp.pad(node_features, ((0, _NPAD - _N), (0, 0)))
    ea_pad = jnp.pad(edge_attr, ((0, _EPAD - _E), (0, 0)))

    srcf = jnp.pad(edge_index[0], (0, _EPAD - _E)).reshape(32, _NSUB, _SUB)
    src = jnp.pad(srcf, ((0, 0), (0, 16 - _NSUB), (0, 0)))

    dstf = jnp.pad(edge_index[1], (0, _EPAD - _E))
    bases = jnp.array([[0], [_QTR], [2 * _QTR], [3 * _QTR]], jnp.int32)
    rel = dstf[None, :] - bases
    routed = jnp.where((rel >= 0) & (rel < _QTR), rel, _QTR)
    dst = routed.astype(jnp.int32).reshape(2, 2, 16, _SROWS, _SUB)
    dst = dst.reshape(64, _SROWS, _SUB)

    t2 = W_e2.reshape(_DH * _DH, _DH).astype(jnp.bfloat16)
    bm = b_e2.reshape(_DH, _DH)
    eye = jnp.eye(_DH, dtype=jnp.bfloat16)
    ones_row = jnp.ones((1, _DH), jnp.bfloat16)
    rmat = jnp.kron(eye, ones_row)
    smat = jnp.kron(ones_row, eye)
    wt = W_msg[:_DH]
    wb = W_msg[_DH:]
    zeros_stripe = jnp.zeros((_ZSTR, _DP), f32)

    sc_gather, sc_scatter = _sc_kernels()
    out = _dense_relu(x_pad, W_in, b_in, pad=True)
    h = _dense_relu(ea_pad, W_e1, b_e1, pad=False, out_dtype=jnp.bfloat16)
    for step in range(3):
        xe = sc_gather(out, src)
        msg = _msg_tc(xe, h, t2, bm, rmat, smat)
        parts = sc_scatter(msg, dst, zeros_stripe)
        agg = parts.reshape(_NPAD, _DP)
        out = _update_tc(agg, out, W_root, conv_bias, wt, wb, b_msg,
                         x_pad, last=(step == 2))
    return out[:_N, :_DH]


# speedup vs baseline: 1.9350x; 1.7442x over previous
import functools

import jax
import jax.numpy as jnp
from jax import lax
from jax.experimental import pallas as pl
from jax.experimental.pallas import tpu as pltpu
from jax.experimental.pallas import tpu_sc as plsc

_N = 20000
_E = 40000
_DH = 64
_DP = 128
_NPAD = 20480
_EPAD = 40960
_TILE = 512
_SUB = 128
_WCHUNK = 1280
_NSUB = _WCHUNK // _SUB
_GROW = 640
_QTR = _NPAD // 4
_SCHUNK = _EPAD // 16
_SROWS = _SCHUNK // _SUB
_ACC = 5248
_ZSTR = _ACC // 16


def _gather_body(nodes_hbm, idx_hbm, out_hbm, idx_v, rows_v, sem):
    wid = lax.axis_index("s") * 2 + lax.axis_index("c")
    pltpu.sync_copy(idx_hbm.at[wid], idx_v)
    for g in range(_WCHUNK // _GROW):
        descs = [
            pltpu.async_copy(
                nodes_hbm.at[idx_v.at[g * 5 + j]],
                rows_v.at[pl.ds(j * _SUB, _SUB)],
                sem,
            )
            for j in range(5)
        ]
        for d in descs:
            d.wait()
        pltpu.sync_copy(
            rows_v, out_hbm.at[pl.ds(wid * _WCHUNK + g * _GROW, _GROW)])


def _scatter_body(msg_hbm, idx_hbm, zeros_hbm, out_hbm, idx_v, rows_v, acc_sh):
    c = lax.axis_index("c")
    s = lax.axis_index("s")
    stripe = _QTR // 16
    for p in range(2):
        pltpu.sync_copy(idx_hbm.at[p * 32 + c * 16 + s], idx_v)
        pltpu.sync_copy(zeros_hbm, acc_sh.at[pl.ds(s * _ZSTR, _ZSTR)])
        plsc.subcore_barrier()
        for g in range(_SCHUNK // _GROW):
            pltpu.sync_copy(
                msg_hbm.at[pl.ds(s * _SCHUNK + g * _GROW, _GROW)], rows_v)
            for j in range(5):
                pltpu.sync_copy(
                    rows_v.at[pl.ds(j * _SUB, _SUB)],
                    acc_sh.at[idx_v.at[g * 5 + j]],
                    add=True,
                )
        plsc.subcore_barrier()
        pltpu.sync_copy(
            acc_sh.at[pl.ds(s * stripe, stripe)],
            out_hbm.at[p * 2 + c, pl.ds(s * stripe, stripe)],
        )
        plsc.subcore_barrier()


@functools.lru_cache(maxsize=None)
def _sc_kernels():
    mesh = plsc.VectorSubcoreMesh(core_axis_name="c", subcore_axis_name="s")
    gather = pl.kernel(
        _gather_body,
        out_type=jax.ShapeDtypeStruct((_EPAD, _DP), jnp.float32),
        mesh=mesh,
        scratch_types=[
            pltpu.VMEM((16, _SUB), jnp.int32),
            pltpu.VMEM((_GROW, _DP), jnp.float32),
            pltpu.SemaphoreType.DMA,
        ],
    )
    scatter = pl.kernel(
        _scatter_body,
        out_type=jax.ShapeDtypeStruct((4, _QTR, _DP), jnp.float32),
        mesh=mesh,
        scratch_types=[
            pltpu.VMEM((_SROWS, _SUB), jnp.int32),
            pltpu.VMEM((_GROW, _DP), jnp.float32),
            pltpu.VMEM_SHARED((_ACC, _DP), jnp.float32),
        ],
    )
    return gather, scatter


def _dense_relu_body(x_ref, w_ref, b_ref, o_ref, *, pad):
    y = jnp.dot(x_ref[...], w_ref[...], preferred_element_type=jnp.float32)
    y = jnp.maximum(y + b_ref[...], 0.0)
    y = y.astype(o_ref.dtype)
    if pad:
        y = jnp.concatenate(
            [y, jnp.zeros((y.shape[0], _DP - _DH), o_ref.dtype)], axis=1)
    o_ref[...] = y


def _dense_relu(x, w, b, pad, out_dtype=jnp.float32):
    rows, din = x.shape
    dout = _DP if pad else _DH
    return pl.pallas_call(
        functools.partial(_dense_relu_body, pad=pad),
        grid=(rows // _TILE,),
        in_specs=[
            pl.BlockSpec((_TILE, din), lambda i: (i, 0)),
            pl.BlockSpec((din, _DH), lambda i: (0, 0)),
            pl.BlockSpec((1, _DH), lambda i: (0, 0)),
        ],
        out_specs=pl.BlockSpec((_TILE, dout), lambda i: (i, 0)),
        out_shape=jax.ShapeDtypeStruct((rows, dout), out_dtype),
    )(x, w, b.reshape(1, _DH))


def _msg_body(x_ref, h_ref, t2_ref, bm_ref, o_ref):
    x = x_ref[:, :_DH]
    xt = jnp.transpose(x).astype(jnp.bfloat16)
    ht = jnp.transpose(h_ref[...])
    pmat_t = (ht[:, None, :] * xt[None, :, :]).reshape(_DH * _DH, _TILE)
    msg_t = lax.dot_general(t2_ref[...], pmat_t, (((0,), (0,)), ((), ())),
                            preferred_element_type=jnp.float32)
    msg = msg_t.T
    msg = msg + jnp.dot(x, bm_ref[...], preferred_element_type=jnp.float32)
    eid = pl.program_id(0) * _TILE + lax.broadcasted_iota(
        jnp.int32, (_TILE, 1), 0)
    msg = jnp.where(eid < _E, msg, 0.0)
    o_ref[...] = jnp.concatenate(
        [msg, jnp.zeros((_TILE, _DP - _DH), jnp.float32)], axis=1)


def _msg_tc(xe, h, t2, bm):
    return pl.pallas_call(
        _msg_body,
        grid=(_EPAD // _TILE,),
        in_specs=[
            pl.BlockSpec((_TILE, _DP), lambda i: (i, 0)),
            pl.BlockSpec((_TILE, _DH), lambda i: (i, 0)),
            pl.BlockSpec((_DH * _DH, _DH), lambda i: (0, 0)),
            pl.BlockSpec((_DH, _DH), lambda i: (0, 0)),
        ],
        out_specs=pl.BlockSpec((_TILE, _DP), lambda i: (i, 0)),
        out_shape=jax.ShapeDtypeStruct((_EPAD, _DP), jnp.float32),
    )(xe, h, t2, bm)


def _update_body(agg_ref, out_ref, wr_ref, cb_ref, wt_ref, wb_ref, bm_ref,
                 init_ref, o_ref, *, last):
    agg = agg_ref[:, :_DH]
    out = out_ref[:, :_DH]
    conv = agg + jnp.dot(out, wr_ref[...],
                         preferred_element_type=jnp.float32) + cb_ref[...]
    m = jnp.maximum(conv, 0.0)
    new = (jnp.dot(m, wt_ref[...], preferred_element_type=jnp.float32)
           + jnp.dot(out, wb_ref[...], preferred_element_type=jnp.float32)
           + bm_ref[...])
    if last:
        new = new + init_ref[...]
    o_ref[...] = jnp.concatenate(
        [new, jnp.zeros((_TILE, _DP - _DH), jnp.float32)], axis=1)


def _update_tc(agg, out, wr, cb, wt, wb, bm, init, last):
    full = lambda i: (0, 0)
    return pl.pallas_call(
        functools.partial(_update_body, last=last),
        grid=(_NPAD // _TILE,),
        in_specs=[
            pl.BlockSpec((_TILE, _DP), lambda i: (i, 0)),
            pl.BlockSpec((_TILE, _DP), lambda i: (i, 0)),
            pl.BlockSpec((_DH, _DH), full),
            pl.BlockSpec((1, _DH), full),
            pl.BlockSpec((_DH, _DH), full),
            pl.BlockSpec((_DH, _DH), full),
            pl.BlockSpec((1, _DH), full),
            pl.BlockSpec((_TILE, _DH), lambda i: (i, 0)),
        ],
        out_specs=pl.BlockSpec((_TILE, _DP), lambda i: (i, 0)),
        out_shape=jax.ShapeDtypeStruct((_NPAD, _DP), jnp.float32),
    )(agg, out, wr, cb.reshape(1, _DH), wt, wb, bm.reshape(1, _DH), init)


def kernel(node_features, edge_attr, edge_index, W_in, b_in, W_msg, b_msg,
           W_e1, b_e1, W_e2, b_e2, W_root, conv_bias):
    f32 = jnp.float32
    x_pad = jnp.pad(node_features, ((0, _NPAD - _N), (0, 0)))
    ea_pad = jnp.pad(edge_attr, ((0, _EPAD - _E), (0, 0)))

    srcf = jnp.pad(edge_index[0], (0, _EPAD - _E)).reshape(32, _NSUB, _SUB)
    src = jnp.pad(srcf, ((0, 0), (0, 16 - _NSUB), (0, 0)))

    dstf = jnp.pad(edge_index[1], (0, _EPAD - _E))
    bases = jnp.array([[0], [_QTR], [2 * _QTR], [3 * _QTR]], jnp.int32)
    rel = dstf[None, :] - bases
    routed = jnp.where((rel >= 0) & (rel < _QTR), rel, _QTR)
    dst = routed.astype(jnp.int32).reshape(2, 2, 16, _SROWS, _SUB)
    dst = dst.reshape(64, _SROWS, _SUB)

    t2 = W_e2.reshape(_DH * _DH, _DH).astype(jnp.bfloat16)
    bm = b_e2.reshape(_DH, _DH)
    wt = W_msg[:_DH]
    wb = W_msg[_DH:]
    zeros_stripe = jnp.zeros((_ZSTR, _DP), f32)

    sc_gather, sc_scatter = _sc_kernels()
    out = _dense_relu(x_pad, W_in, b_in, pad=True)
    h = _dense_relu(ea_pad, W_e1, b_e1, pad=False, out_dtype=jnp.bfloat16)
    for step in range(3):
        xe = sc_gather(out, src)
        msg = _msg_tc(xe, h, t2, bm)
        parts = sc_scatter(msg, dst, zeros_stripe)
        agg = parts.reshape(_NPAD, _DP)
        out = _update_tc(agg, out, W_root, conv_bias, wt, wb, b_msg,
                         x_pad, last=(step == 2))
    return out[:_N, :_DH]


# speedup vs baseline: 2.0203x; 1.0441x over previous
import functools

import jax
import jax.numpy as jnp
from jax import lax
from jax.experimental import pallas as pl
from jax.experimental.pallas import tpu as pltpu
from jax.experimental.pallas import tpu_sc as plsc

_N = 20000
_E = 40000
_DH = 64
_DP = 128
_NPAD = 20480
_EPAD = 40960
_TILE = 512
_SUB = 128
_WCHUNK = 1280
_NSUB = _WCHUNK // _SUB
_SGR = 256
_QTR = _NPAD // 4
_SCHUNK = _EPAD // 16
_SROWS = _SCHUNK // _SUB
_ACC = 5248
_ZSTR = _ACC // 16


def _gather_body(nodes_hbm, idx_hbm, out_hbm, idx_v, rows_a, rows_b,
                 sg_a, sg_b, so_a, so_b):
    wid = lax.axis_index("s") * 2 + lax.axis_index("c")
    pltpu.sync_copy(idx_hbm.at[wid], idx_v)
    bufs = (rows_a, rows_b)
    gsems = (sg_a, sg_b)
    osems = (so_a, so_b)
    ng = _WCHUNK // _SGR
    gats = [None] * ng
    outs = [None] * ng

    def issue_gather(g):
        buf, sem = bufs[g % 2], gsems[g % 2]
        gats[g] = [
            pltpu.async_copy(
                nodes_hbm.at[idx_v.at[g * 2 + j]],
                buf.at[pl.ds(j * _SUB, _SUB)], sem)
            for j in range(2)
        ]

    issue_gather(0)
    for g in range(ng):
        for d in gats[g]:
            d.wait()
        if g + 1 < ng:
            if g - 1 >= 0:
                outs[g - 1].wait()
            issue_gather(g + 1)
        outs[g] = pltpu.async_copy(
            bufs[g % 2],
            out_hbm.at[pl.ds(wid * _WCHUNK + g * _SGR, _SGR)],
            osems[g % 2])
    outs[ng - 2].wait()
    outs[ng - 1].wait()


def _scatter_body(msg_hbm, idx_hbm, zeros_hbm, out_hbm, idx_v, rows_a,
                  rows_b, acc_sh, sl_a, sl_b, ss_a, ss_b):
    c = lax.axis_index("c")
    s = lax.axis_index("s")
    stripe = _QTR // 16
    bufs = (rows_a, rows_b)
    lsems = (sl_a, sl_b)
    ssems = (ss_a, ss_b)
    ng = _SCHUNK // _SGR
    for p in range(2):
        pltpu.sync_copy(idx_hbm.at[p * 32 + c * 16 + s], idx_v)
        pltpu.sync_copy(zeros_hbm, acc_sh.at[pl.ds(s * _ZSTR, _ZSTR)])
        plsc.subcore_barrier()
        loads = [None] * ng
        scats = [None] * ng

        def issue_load(g):
            loads[g] = pltpu.async_copy(
                msg_hbm.at[pl.ds(s * _SCHUNK + g * _SGR, _SGR)],
                bufs[g % 2], lsems[g % 2])

        issue_load(0)
        for g in range(ng):
            loads[g].wait()
            scats[g] = [
                pltpu.async_copy(
                    bufs[g % 2].at[pl.ds(j * _SUB, _SUB)],
                    acc_sh.at[idx_v.at[g * 2 + j]],
                    ssems[g % 2], add=True)
                for j in range(2)
            ]
            if g + 1 < ng:
                if g - 1 >= 0:
                    for d in scats[g - 1]:
                        d.wait()
                issue_load(g + 1)
        for d in scats[ng - 2]:
            d.wait()
        for d in scats[ng - 1]:
            d.wait()
        plsc.subcore_barrier()
        pltpu.sync_copy(
            acc_sh.at[pl.ds(s * stripe, stripe)],
            out_hbm.at[p * 2 + c, pl.ds(s * stripe, stripe)],
        )
        plsc.subcore_barrier()


@functools.lru_cache(maxsize=None)
def _sc_kernels():
    mesh = plsc.VectorSubcoreMesh(core_axis_name="c", subcore_axis_name="s")
    gather = pl.kernel(
        _gather_body,
        out_type=jax.ShapeDtypeStruct((_EPAD, _DP), jnp.float32),
        mesh=mesh,
        scratch_types=[
            pltpu.VMEM((16, _SUB), jnp.int32),
            pltpu.VMEM((_SGR, _DP), jnp.float32),
            pltpu.VMEM((_SGR, _DP), jnp.float32),
            pltpu.SemaphoreType.DMA,
            pltpu.SemaphoreType.DMA,
            pltpu.SemaphoreType.DMA,
            pltpu.SemaphoreType.DMA,
        ],
    )
    scatter = pl.kernel(
        _scatter_body,
        out_type=jax.ShapeDtypeStruct((4, _QTR, _DP), jnp.float32),
        mesh=mesh,
        scratch_types=[
            pltpu.VMEM((_SROWS, _SUB), jnp.int32),
            pltpu.VMEM((_SGR, _DP), jnp.float32),
            pltpu.VMEM((_SGR, _DP), jnp.float32),
            pltpu.VMEM_SHARED((_ACC, _DP), jnp.float32),
            pltpu.SemaphoreType.DMA,
            pltpu.SemaphoreType.DMA,
            pltpu.SemaphoreType.DMA,
            pltpu.SemaphoreType.DMA,
        ],
    )
    return gather, scatter


def _dense_relu_body(x_ref, w_ref, b_ref, o_ref, *, pad):
    y = jnp.dot(x_ref[...], w_ref[...], preferred_element_type=jnp.float32)
    y = jnp.maximum(y + b_ref[...], 0.0)
    y = y.astype(o_ref.dtype)
    if pad:
        y = jnp.concatenate(
            [y, jnp.zeros((y.shape[0], _DP - _DH), o_ref.dtype)], axis=1)
    o_ref[...] = y


def _dense_relu(x, w, b, pad, out_dtype=jnp.float32):
    rows, din = x.shape
    dout = _DP if pad else _DH
    return pl.pallas_call(
        functools.partial(_dense_relu_body, pad=pad),
        grid=(rows // _TILE,),
        in_specs=[
            pl.BlockSpec((_TILE, din), lambda i: (i, 0)),
            pl.BlockSpec((din, _DH), lambda i: (0, 0)),
            pl.BlockSpec((1, _DH), lambda i: (0, 0)),
        ],
        out_specs=pl.BlockSpec((_TILE, dout), lambda i: (i, 0)),
        out_shape=jax.ShapeDtypeStruct((rows, dout), out_dtype),
    )(x, w, b.reshape(1, _DH))


def _msg_body(x_ref, h_ref, t2_ref, bm_ref, o_ref):
    x = x_ref[:, :_DH]
    xt = jnp.transpose(x).astype(jnp.bfloat16)
    ht = jnp.transpose(h_ref[...])
    pmat_t = (ht[:, None, :] * xt[None, :, :]).reshape(_DH * _DH, _TILE)
    msg_t = lax.dot_general(t2_ref[...], pmat_t, (((0,), (0,)), ((), ())),
                            preferred_element_type=jnp.float32)
    msg = msg_t.T
    msg = msg + jnp.dot(x, bm_ref[...], preferred_element_type=jnp.float32)
    eid = pl.program_id(0) * _TILE + lax.broadcasted_iota(
        jnp.int32, (_TILE, 1), 0)
    msg = jnp.where(eid < _E, msg, 0.0)
    o_ref[...] = jnp.concatenate(
        [msg, jnp.zeros((_TILE, _DP - _DH), jnp.float32)], axis=1)


def _msg_tc(xe, h, t2, bm):
    return pl.pallas_call(
        _msg_body,
        grid=(_EPAD // _TILE,),
        in_specs=[
            pl.BlockSpec((_TILE, _DP), lambda i: (i, 0)),
            pl.BlockSpec((_TILE, _DH), lambda i: (i, 0)),
            pl.BlockSpec((_DH * _DH, _DH), lambda i: (0, 0)),
            pl.BlockSpec((_DH, _DH), lambda i: (0, 0)),
        ],
        out_specs=pl.BlockSpec((_TILE, _DP), lambda i: (i, 0)),
        out_shape=jax.ShapeDtypeStruct((_EPAD, _DP), jnp.float32),
    )(xe, h, t2, bm)


def _update_body(agg_ref, out_ref, wr_ref, cb_ref, wt_ref, wb_ref, bm_ref,
                 init_ref, o_ref, *, last):
    agg = agg_ref[:, :_DH]
    out = out_ref[:, :_DH]
    conv = agg + jnp.dot(out, wr_ref[...],
                         preferred_element_type=jnp.float32) + cb_ref[...]
    m = jnp.maximum(conv, 0.0)
    new = (jnp.dot(m, wt_ref[...], preferred_element_type=jnp.float32)
           + jnp.dot(out, wb_ref[...], preferred_element_type=jnp.float32)
           + bm_ref[...])
    if last:
        new = new + init_ref[...]
    o_ref[...] = jnp.concatenate(
        [new, jnp.zeros((_TILE, _DP - _DH), jnp.float32)], axis=1)


def _update_tc(agg, out, wr, cb, wt, wb, bm, init, last):
    full = lambda i: (0, 0)
    return pl.pallas_call(
        functools.partial(_update_body, last=last),
        grid=(_NPAD // _TILE,),
        in_specs=[
            pl.BlockSpec((_TILE, _DP), lambda i: (i, 0)),
            pl.BlockSpec((_TILE, _DP), lambda i: (i, 0)),
            pl.BlockSpec((_DH, _DH), full),
            pl.BlockSpec((1, _DH), full),
            pl.BlockSpec((_DH, _DH), full),
            pl.BlockSpec((_DH, _DH), full),
            pl.BlockSpec((1, _DH), full),
            pl.BlockSpec((_TILE, _DH), lambda i: (i, 0)),
        ],
        out_specs=pl.BlockSpec((_TILE, _DP), lambda i: (i, 0)),
        out_shape=jax.ShapeDtypeStruct((_NPAD, _DP), jnp.float32),
    )(agg, out, wr, cb.reshape(1, _DH), wt, wb, bm.reshape(1, _DH), init)


def kernel(node_features, edge_attr, edge_index, W_in, b_in, W_msg, b_msg,
           W_e1, b_e1, W_e2, b_e2, W_root, conv_bias):
    f32 = jnp.float32
    x_pad = jnp.pad(node_features, ((0, _NPAD - _N), (0, 0)))
    ea_pad = jnp.pad(edge_attr, ((0, _EPAD - _E), (0, 0)))

    srcf = jnp.pad(edge_index[0], (0, _EPAD - _E)).reshape(32, _NSUB, _SUB)
    src = jnp.pad(srcf, ((0, 0), (0, 16 - _NSUB), (0, 0)))

    dstf = jnp.pad(edge_index[1], (0, _EPAD - _E))
    bases = jnp.array([[0], [_QTR], [2 * _QTR], [3 * _QTR]], jnp.int32)
    rel = dstf[None, :] - bases
    routed = jnp.where((rel >= 0) & (rel < _QTR), rel, _QTR)
    dst = routed.astype(jnp.int32).reshape(2, 2, 16, _SROWS, _SUB)
    dst = dst.reshape(64, _SROWS, _SUB)

    t2 = W_e2.reshape(_DH * _DH, _DH).astype(jnp.bfloat16)
    bm = b_e2.reshape(_DH, _DH)
    wt = W_msg[:_DH]
    wb = W_msg[_DH:]
    zeros_stripe = jnp.zeros((_ZSTR, _DP), f32)

    sc_gather, sc_scatter = _sc_kernels()
    out = _dense_relu(x_pad, W_in, b_in, pad=True)
    h = _dense_relu(ea_pad, W_e1, b_e1, pad=False, out_dtype=jnp.bfloat16)
    for step in range(3):
        xe = sc_gather(out, src)
        msg = _msg_tc(xe, h, t2, bm)
        parts = sc_scatter(msg, dst, zeros_stripe)
        agg = parts.reshape(_NPAD, _DP)
        out = _update_tc(agg, out, W_root, conv_bias, wt, wb, b_msg,
                         x_pad, last=(step == 2))
    return out[:_N, :_DH]


# speedup vs baseline: 2.1848x; 1.0814x over previous
import functools

import jax
import jax.numpy as jnp
from jax import lax
from jax.experimental import pallas as pl
from jax.experimental.pallas import tpu as pltpu
from jax.experimental.pallas import tpu_sc as plsc

_N = 20000
_E = 40000
_DH = 64
_DP = 128
_NPAD = 20480
_EPAD = 40960
_TILE = 512
_SUB = 128
_WCHUNK = 1280
_NSUB = _WCHUNK // _SUB
_SGR = 256
_QTR = _NPAD // 4
_SCHUNK = _EPAD // 16
_SROWS = _SCHUNK // _SUB
_ACC = 5248
_ZSTR = _ACC // 16


def _gather_body(nodes_hbm, idx_hbm, out_hbm, idx_v, rows_a, rows_b,
                 sg_a, sg_b, so_a, so_b):
    wid = lax.axis_index("s") * 2 + lax.axis_index("c")
    pltpu.sync_copy(idx_hbm.at[wid], idx_v)
    bufs = (rows_a, rows_b)
    gsems = (sg_a, sg_b)
    osems = (so_a, so_b)
    ng = _WCHUNK // _SGR
    gats = [None] * ng
    outs = [None] * ng

    def issue_gather(g):
        buf, sem = bufs[g % 2], gsems[g % 2]
        gats[g] = [
            pltpu.async_copy(
                nodes_hbm.at[idx_v.at[g * 2 + j]],
                buf.at[pl.ds(j * _SUB, _SUB)], sem)
            for j in range(2)
        ]

    issue_gather(0)
    for g in range(ng):
        for d in gats[g]:
            d.wait()
        if g + 1 < ng:
            if g - 1 >= 0:
                outs[g - 1].wait()
            issue_gather(g + 1)
        outs[g] = pltpu.async_copy(
            bufs[g % 2],
            out_hbm.at[pl.ds(wid * _WCHUNK + g * _SGR, _SGR)],
            osems[g % 2])
    outs[ng - 2].wait()
    outs[ng - 1].wait()


def _scatter_body(msg_hbm, idx_hbm, zeros_hbm, out_hbm, idx_v, rows_a,
                  rows_b, acc_sh, sl_a, sl_b, ss_a, ss_b):
    c = lax.axis_index("c")
    s = lax.axis_index("s")
    stripe = _QTR // 16
    bufs = (rows_a, rows_b)
    lsems = (sl_a, sl_b)
    ssems = (ss_a, ss_b)
    ng = _SCHUNK // _SGR
    for p in range(2):
        pltpu.sync_copy(idx_hbm.at[p * 32 + c * 16 + s], idx_v)
        pltpu.sync_copy(zeros_hbm, acc_sh.at[pl.ds(s * _ZSTR, _ZSTR)])
        plsc.subcore_barrier()
        loads = [None] * ng
        scats = [None] * ng

        def issue_load(g):
            loads[g] = pltpu.async_copy(
                msg_hbm.at[pl.ds(s * _SCHUNK + g * _SGR, _SGR)],
                bufs[g % 2], lsems[g % 2])

        issue_load(0)
        for g in range(ng):
            loads[g].wait()
            scats[g] = [
                pltpu.async_copy(
                    bufs[g % 2].at[pl.ds(j * _SUB, _SUB)],
                    acc_sh.at[idx_v.at[g * 2 + j]],
                    ssems[g % 2], add=True)
                for j in range(2)
            ]
            if g + 1 < ng:
                if g - 1 >= 0:
                    for d in scats[g - 1]:
                        d.wait()
                issue_load(g + 1)
        for d in scats[ng - 2]:
            d.wait()
        for d in scats[ng - 1]:
            d.wait()
        plsc.subcore_barrier()
        pltpu.sync_copy(
            acc_sh.at[pl.ds(s * stripe, stripe)],
            out_hbm.at[p * 2 + c, pl.ds(s * stripe, stripe)],
        )
        plsc.subcore_barrier()


@functools.lru_cache(maxsize=None)
def _sc_kernels():
    mesh = plsc.VectorSubcoreMesh(core_axis_name="c", subcore_axis_name="s")
    gather = pl.kernel(
        _gather_body,
        out_type=jax.ShapeDtypeStruct((_EPAD, _DP), jnp.float32),
        mesh=mesh,
        scratch_types=[
            pltpu.VMEM((16, _SUB), jnp.int32),
            pltpu.VMEM((_SGR, _DP), jnp.float32),
            pltpu.VMEM((_SGR, _DP), jnp.float32),
            pltpu.SemaphoreType.DMA,
            pltpu.SemaphoreType.DMA,
            pltpu.SemaphoreType.DMA,
            pltpu.SemaphoreType.DMA,
        ],
    )
    scatter = pl.kernel(
        _scatter_body,
        out_type=jax.ShapeDtypeStruct((4, _QTR, _DP), jnp.float32),
        mesh=mesh,
        scratch_types=[
            pltpu.VMEM((_SROWS, _SUB), jnp.int32),
            pltpu.VMEM((_SGR, _DP), jnp.float32),
            pltpu.VMEM((_SGR, _DP), jnp.float32),
            pltpu.VMEM_SHARED((_ACC, _DP), jnp.float32),
            pltpu.SemaphoreType.DMA,
            pltpu.SemaphoreType.DMA,
            pltpu.SemaphoreType.DMA,
            pltpu.SemaphoreType.DMA,
        ],
    )
    return gather, scatter


def _bpad(y):
    return jnp.concatenate(
        [y, jnp.zeros((y.shape[0], _DP - _DH), jnp.float32)], axis=1)


def _dense_relu_body(x_ref, w_ref, b_ref, o_ref):
    y = jnp.dot(x_ref[...], w_ref[...], preferred_element_type=jnp.float32)
    y = jnp.maximum(y + b_ref[...], 0.0)
    o_ref[...] = y.astype(o_ref.dtype)


def _dense_relu(x, w, b, out_dtype=jnp.float32):
    rows, din = x.shape
    return pl.pallas_call(
        _dense_relu_body,
        grid=(rows // _TILE,),
        in_specs=[
            pl.BlockSpec((_TILE, din), lambda i: (i, 0)),
            pl.BlockSpec((din, _DH), lambda i: (0, 0)),
            pl.BlockSpec((1, _DH), lambda i: (0, 0)),
        ],
        out_specs=pl.BlockSpec((_TILE, _DH), lambda i: (i, 0)),
        out_shape=jax.ShapeDtypeStruct((rows, _DH), out_dtype),
    )(x, w, b.reshape(1, _DH))


def _init_body(x_ref, w_ref, b_ref, o_ref, ob_ref):
    y = jnp.dot(x_ref[...], w_ref[...], preferred_element_type=jnp.float32)
    y = jnp.maximum(y + b_ref[...], 0.0)
    o_ref[...] = y
    ob_ref[...] = _bpad(y)


def _init_tc(x, w, b):
    return pl.pallas_call(
        _init_body,
        grid=(_NPAD // _TILE,),
        in_specs=[
            pl.BlockSpec((_TILE, _DH), lambda i: (i, 0)),
            pl.BlockSpec((_DH, _DH), lambda i: (0, 0)),
            pl.BlockSpec((1, _DH), lambda i: (0, 0)),
        ],
        out_specs=[
            pl.BlockSpec((_TILE, _DH), lambda i: (i, 0)),
            pl.BlockSpec((_TILE, _DP), lambda i: (i, 0)),
        ],
        out_shape=[
            jax.ShapeDtypeStruct((_NPAD, _DH), jnp.float32),
            jax.ShapeDtypeStruct((_NPAD, _DP), jnp.float32),
        ],
    )(x, w, b.reshape(1, _DH))


def _msg_body(x_ref, h_ref, t2_ref, bm_ref, o_ref):
    x = x_ref[:, :_DH]
    xt = jnp.transpose(x).astype(jnp.bfloat16)
    ht = jnp.transpose(h_ref[...])
    pmat_t = (ht[:, None, :] * xt[None, :, :]).reshape(_DH * _DH, _TILE)
    msg_t = lax.dot_general(t2_ref[...], pmat_t, (((0,), (0,)), ((), ())),
                            preferred_element_type=jnp.float32)
    msg = msg_t.T
    msg = msg + jnp.dot(x, bm_ref[...], preferred_element_type=jnp.float32)
    eid = pl.program_id(0) * _TILE + lax.broadcasted_iota(
        jnp.int32, (_TILE, 1), 0)
    msg = jnp.where(eid < _E, msg, 0.0)
    o_ref[...] = jnp.concatenate(
        [msg, jnp.zeros((_TILE, _DP - _DH), jnp.float32)], axis=1)


def _msg_tc(xe, h, t2, bm):
    return pl.pallas_call(
        _msg_body,
        grid=(_EPAD // _TILE,),
        in_specs=[
            pl.BlockSpec((_TILE, _DP), lambda i: (i, 0)),
            pl.BlockSpec((_TILE, _DH), lambda i: (i, 0)),
            pl.BlockSpec((_DH * _DH, _DH), lambda i: (0, 0)),
            pl.BlockSpec((_DH, _DH), lambda i: (0, 0)),
        ],
        out_specs=pl.BlockSpec((_TILE, _DP), lambda i: (i, 0)),
        out_shape=jax.ShapeDtypeStruct((_EPAD, _DP), jnp.float32),
    )(xe, h, t2, bm)


def _update_body(agg_ref, out_ref, wr_ref, cb_ref, wt_ref, wb_ref, bm_ref,
                 init_ref, o_ref, ob_ref, *, last):
    agg = agg_ref[:, :_DH]
    out = out_ref[...]
    conv = agg + jnp.dot(out, wr_ref[...],
                         preferred_element_type=jnp.float32) + cb_ref[...]
    m = jnp.maximum(conv, 0.0)
    new = (jnp.dot(m, wt_ref[...], preferred_element_type=jnp.float32)
           + jnp.dot(out, wb_ref[...], preferred_element_type=jnp.float32)
           + bm_ref[...])
    if last:
        new = new + init_ref[...]
    o_ref[...] = new
    ob_ref[...] = _bpad(new)


def _update_tc(agg, out, wr, cb, wt, wb, bm, init, last):
    full = lambda i: (0, 0)
    return pl.pallas_call(
        functools.partial(_update_body, last=last),
        grid=(_NPAD // _TILE,),
        in_specs=[
            pl.BlockSpec((_TILE, _DP), lambda i: (i, 0)),
            pl.BlockSpec((_TILE, _DH), lambda i: (i, 0)),
            pl.BlockSpec((_DH, _DH), full),
            pl.BlockSpec((1, _DH), full),
            pl.BlockSpec((_DH, _DH), full),
            pl.BlockSpec((_DH, _DH), full),
            pl.BlockSpec((1, _DH), full),
            pl.BlockSpec((_TILE, _DH), lambda i: (i, 0)),
        ],
        out_specs=[
            pl.BlockSpec((_TILE, _DH), lambda i: (i, 0)),
            pl.BlockSpec((_TILE, _DP), lambda i: (i, 0)),
        ],
        out_shape=[
            jax.ShapeDtypeStruct((_NPAD, _DH), jnp.float32),
            jax.ShapeDtypeStruct((_NPAD, _DP), jnp.float32),
        ],
    )(agg, out, wr, cb.reshape(1, _DH), wt, wb, bm.reshape(1, _DH), init)


def kernel(node_features, edge_attr, edge_index, W_in, b_in, W_msg, b_msg,
           W_e1, b_e1, W_e2, b_e2, W_root, conv_bias):
    f32 = jnp.float32
    x_pad = jnp.pad(node_features, ((0, _NPAD - _N), (0, 0)))
    ea_pad = jnp.pad(edge_attr, ((0, _EPAD - _E), (0, 0)))

    srcf = jnp.pad(edge_index[0], (0, _EPAD - _E)).reshape(32, _NSUB, _SUB)
    src = jnp.pad(srcf, ((0, 0), (0, 16 - _NSUB), (0, 0)))

    dstf = jnp.pad(edge_index[1], (0, _EPAD - _E))
    bases = jnp.array([[0], [_QTR], [2 * _QTR], [3 * _QTR]], jnp.int32)
    rel = dstf[None, :] - bases
    dump = _QTR + (jnp.arange(_EPAD, dtype=jnp.int32) % 128)[None, :]
    routed = jnp.where((rel >= 0) & (rel < _QTR), rel, dump)
    dst = routed.astype(jnp.int32).reshape(2, 2, 16, _SROWS, _SUB)
    dst = dst.reshape(64, _SROWS, _SUB)

    t2 = W_e2.reshape(_DH * _DH, _DH).astype(jnp.bfloat16)
    bm = b_e2.reshape(_DH, _DH)
    wt = W_msg[:_DH]
    wb = W_msg[_DH:]
    zeros_stripe = jnp.zeros((_ZSTR, _DP), f32)

    sc_gather, sc_scatter = _sc_kernels()
    out, outb = _init_tc(x_pad, W_in, b_in)
    h = _dense_relu(ea_pad, W_e1, b_e1, out_dtype=jnp.bfloat16)
    for step in range(3):
        xe = sc_gather(outb, src)
        msg = _msg_tc(xe, h, t2, bm)
        parts = sc_scatter(msg, dst, zeros_stripe)
        agg = parts.reshape(_NPAD, _DP)
        out, outb = _update_tc(agg, out, W_root, conv_bias, wt, wb, b_msg,
                               x_pad, last=(step == 2))
    return out[:_N]


# speedup vs baseline: 2.2711x; 1.0395x over previous
import functools

import jax
import jax.numpy as jnp
from jax import lax
from jax.experimental import pallas as pl
from jax.experimental.pallas import tpu as pltpu
from jax.experimental.pallas import tpu_sc as plsc

_N = 20000
_E = 40000
_DH = 64
_DP = 128
_NPAD = 20480
_EPAD = 40960
_TILE = 512
_SUB = 128
_SGR = 256
_EH = _EPAD // 2
_GCH = _EH // 32
_GNG = _GCH // _SUB
_SCH = _EH // 16
_SNG = _SCH // _SGR
_QTR = _NPAD // 4
_ACC = 5248
_ZSTR = _ACC // 16


def _gather_body(nodes_hbm, idx_hbm, out_hbm, idx_v, rows_a, rows_b,
                 sg_a, sg_b, so_a, so_b):
    wid = lax.axis_index("s") * 2 + lax.axis_index("c")
    pltpu.sync_copy(idx_hbm.at[wid], idx_v)
    bufs = (rows_a, rows_b)
    gsems = (sg_a, sg_b)
    osems = (so_a, so_b)
    ng = _GNG
    gats = [None] * ng
    outs = [None] * ng

    def issue_gather(g):
        gats[g] = pltpu.async_copy(
            nodes_hbm.at[idx_v.at[g]], bufs[g % 2], gsems[g % 2])

    issue_gather(0)
    for g in range(ng):
        gats[g].wait()
        if g + 1 < ng:
            if g - 1 >= 0:
                outs[g - 1].wait()
            issue_gather(g + 1)
        outs[g] = pltpu.async_copy(
            bufs[g % 2],
            out_hbm.at[pl.ds(wid * _GCH + g * _SUB, _SUB)],
            osems[g % 2])
    outs[ng - 2].wait()
    outs[ng - 1].wait()


def _scatter_body(msg_hbm, idx_hbm, zeros_hbm, out_hbm, idx_v, rows_a,
                  rows_b, acc_sh, sl_a, sl_b, ss_a, ss_b):
    c = lax.axis_index("c")
    s = lax.axis_index("s")
    stripe = _QTR // 16
    bufs = (rows_a, rows_b)
    lsems = (sl_a, sl_b)
    ssems = (ss_a, ss_b)
    ng = _SNG
    for p in range(2):
        pltpu.sync_copy(idx_hbm.at[p * 32 + c * 16 + s], idx_v)
        pltpu.sync_copy(zeros_hbm, acc_sh.at[pl.ds(s * _ZSTR, _ZSTR)])
        plsc.subcore_barrier()
        loads = [None] * ng
        scats = [None] * ng

        def issue_load(g):
            loads[g] = pltpu.async_copy(
                msg_hbm.at[pl.ds(s * _SCH + g * _SGR, _SGR)],
                bufs[g % 2], lsems[g % 2])

        issue_load(0)
        for g in range(ng):
            loads[g].wait()
            scats[g] = [
                pltpu.async_copy(
                    bufs[g % 2].at[pl.ds(j * _SUB, _SUB)],
                    acc_sh.at[idx_v.at[g * 2 + j]],
                    ssems[g % 2], add=True)
                for j in range(2)
            ]
            if g + 1 < ng:
                if g - 1 >= 0:
                    for d in scats[g - 1]:
                        d.wait()
                issue_load(g + 1)
        for d in scats[ng - 2]:
            d.wait()
        for d in scats[ng - 1]:
            d.wait()
        plsc.subcore_barrier()
        pltpu.sync_copy(
            acc_sh.at[pl.ds(s * stripe, stripe)],
            out_hbm.at[p * 2 + c, pl.ds(s * stripe, stripe)],
        )
        plsc.subcore_barrier()


@functools.lru_cache(maxsize=None)
def _sc_kernels():
    mesh = plsc.VectorSubcoreMesh(core_axis_name="c", subcore_axis_name="s")
    gather = pl.kernel(
        _gather_body,
        out_type=jax.ShapeDtypeStruct((_EH, _DP), jnp.float32),
        mesh=mesh,
        scratch_types=[
            pltpu.VMEM((8, _SUB), jnp.int32),
            pltpu.VMEM((_SUB, _DP), jnp.float32),
            pltpu.VMEM((_SUB, _DP), jnp.float32),
            pltpu.SemaphoreType.DMA,
            pltpu.SemaphoreType.DMA,
            pltpu.SemaphoreType.DMA,
            pltpu.SemaphoreType.DMA,
        ],
    )
    scatter = pl.kernel(
        _scatter_body,
        out_type=jax.ShapeDtypeStruct((4, _QTR, _DP), jnp.float32),
        mesh=mesh,
        scratch_types=[
            pltpu.VMEM((16, _SUB), jnp.int32),
            pltpu.VMEM((_SGR, _DP), jnp.float32),
            pltpu.VMEM((_SGR, _DP), jnp.float32),
            pltpu.VMEM_SHARED((_ACC, _DP), jnp.float32),
            pltpu.SemaphoreType.DMA,
            pltpu.SemaphoreType.DMA,
            pltpu.SemaphoreType.DMA,
            pltpu.SemaphoreType.DMA,
        ],
    )
    return gather, scatter


def _bpad(y):
    return jnp.concatenate(
        [y, jnp.zeros((y.shape[0], _DP - _DH), jnp.float32)], axis=1)


def _dense_relu_body(x_ref, w_ref, b_ref, o_ref):
    y = jnp.dot(x_ref[...], w_ref[...], preferred_element_type=jnp.float32)
    y = jnp.maximum(y + b_ref[...], 0.0)
    o_ref[...] = y.astype(o_ref.dtype)


def _dense_relu(x, w, b, out_dtype=jnp.float32):
    rows, din = x.shape
    return pl.pallas_call(
        _dense_relu_body,
        grid=(rows // _TILE,),
        in_specs=[
            pl.BlockSpec((_TILE, din), lambda i: (i, 0)),
            pl.BlockSpec((din, _DH), lambda i: (0, 0)),
            pl.BlockSpec((1, _DH), lambda i: (0, 0)),
        ],
        out_specs=pl.BlockSpec((_TILE, _DH), lambda i: (i, 0)),
        out_shape=jax.ShapeDtypeStruct((rows, _DH), out_dtype),
    )(x, w, b.reshape(1, _DH))


def _init_body(x_ref, w_ref, b_ref, o_ref, ob_ref):
    y = jnp.dot(x_ref[...], w_ref[...], preferred_element_type=jnp.float32)
    y = jnp.maximum(y + b_ref[...], 0.0)
    o_ref[...] = y
    ob_ref[...] = _bpad(y)


def _init_tc(x, w, b):
    return pl.pallas_call(
        _init_body,
        grid=(_NPAD // _TILE,),
        in_specs=[
            pl.BlockSpec((_TILE, _DH), lambda i: (i, 0)),
            pl.BlockSpec((_DH, _DH), lambda i: (0, 0)),
            pl.BlockSpec((1, _DH), lambda i: (0, 0)),
        ],
        out_specs=[
            pl.BlockSpec((_TILE, _DH), lambda i: (i, 0)),
            pl.BlockSpec((_TILE, _DP), lambda i: (i, 0)),
        ],
        out_shape=[
            jax.ShapeDtypeStruct((_NPAD, _DH), jnp.float32),
            jax.ShapeDtypeStruct((_NPAD, _DP), jnp.float32),
        ],
    )(x, w, b.reshape(1, _DH))


def _msg_body(x_ref, h_ref, t2_ref, bm_ref, o_ref, *, half):
    x = x_ref[:, :_DH]
    xt = jnp.transpose(x).astype(jnp.bfloat16)
    ht = jnp.transpose(h_ref[...])
    pmat_t = (ht[:, None, :] * xt[None, :, :]).reshape(_DH * _DH, _TILE)
    msg_t = lax.dot_general(t2_ref[...], pmat_t, (((0,), (0,)), ((), ())),
                            preferred_element_type=jnp.float32)
    msg = msg_t.T
    msg = msg + jnp.dot(x, bm_ref[...], preferred_element_type=jnp.float32)
    eid = (half * _EH + pl.program_id(0) * _TILE
           + lax.broadcasted_iota(jnp.int32, (_TILE, 1), 0))
    msg = jnp.where(eid < _E, msg, 0.0)
    o_ref[...] = jnp.concatenate(
        [msg, jnp.zeros((_TILE, _DP - _DH), jnp.float32)], axis=1)


def _msg_tc(xe, h_half, t2, bm, half):
    return pl.pallas_call(
        functools.partial(_msg_body, half=half),
        grid=(_EH // _TILE,),
        in_specs=[
            pl.BlockSpec((_TILE, _DP), lambda i: (i, 0)),
            pl.BlockSpec((_TILE, _DH), lambda i: (i, 0)),
            pl.BlockSpec((_DH * _DH, _DH), lambda i: (0, 0)),
            pl.BlockSpec((_DH, _DH), lambda i: (0, 0)),
        ],
        out_specs=pl.BlockSpec((_TILE, _DP), lambda i: (i, 0)),
        out_shape=jax.ShapeDtypeStruct((_EH, _DP), jnp.float32),
    )(xe, h_half, t2, bm)


def _update_body(agg_ref, agg2_ref, out_ref, wr_ref, cb_ref, wt_ref, wb_ref,
                 bm_ref, init_ref, o_ref, ob_ref, *, last):
    agg = agg_ref[:, :_DH] + agg2_ref[:, :_DH]
    out = out_ref[...]
    conv = agg + jnp.dot(out, wr_ref[...],
                         preferred_element_type=jnp.float32) + cb_ref[...]
    m = jnp.maximum(conv, 0.0)
    new = (jnp.dot(m, wt_ref[...], preferred_element_type=jnp.float32)
           + jnp.dot(out, wb_ref[...], preferred_element_type=jnp.float32)
           + bm_ref[...])
    if last:
        new = new + init_ref[...]
    o_ref[...] = new
    ob_ref[...] = _bpad(new)


def _update_tc(agg, agg2, out, wr, cb, wt, wb, bm, init, last):
    full = lambda i: (0, 0)
    return pl.pallas_call(
        functools.partial(_update_body, last=last),
        grid=(_NPAD // _TILE,),
        in_specs=[
            pl.BlockSpec((_TILE, _DP), lambda i: (i, 0)),
            pl.BlockSpec((_TILE, _DP), lambda i: (i, 0)),
            pl.BlockSpec((_TILE, _DH), lambda i: (i, 0)),
            pl.BlockSpec((_DH, _DH), full),
            pl.BlockSpec((1, _DH), full),
            pl.BlockSpec((_DH, _DH), full),
            pl.BlockSpec((_DH, _DH), full),
            pl.BlockSpec((1, _DH), full),
            pl.BlockSpec((_TILE, _DH), lambda i: (i, 0)),
        ],
        out_specs=[
            pl.BlockSpec((_TILE, _DH), lambda i: (i, 0)),
            pl.BlockSpec((_TILE, _DP), lambda i: (i, 0)),
        ],
        out_shape=[
            jax.ShapeDtypeStruct((_NPAD, _DH), jnp.float32),
            jax.ShapeDtypeStruct((_NPAD, _DP), jnp.float32),
        ],
    )(agg, agg2, out, wr, cb.reshape(1, _DH), wt, wb, bm.reshape(1, _DH),
      init)


def kernel(node_features, edge_attr, edge_index, W_in, b_in, W_msg, b_msg,
           W_e1, b_e1, W_e2, b_e2, W_root, conv_bias):
    f32 = jnp.float32
    x_pad = jnp.pad(node_features, ((0, _NPAD - _N), (0, 0)))
    ea_pad = jnp.pad(edge_attr, ((0, _EPAD - _E), (0, 0)))

    srcf = jnp.pad(edge_index[0], (0, _EPAD - _E)).reshape(2, 32, _GNG, _SUB)
    src_h = [jnp.pad(srcf[hh], ((0, 0), (0, 8 - _GNG), (0, 0)))
             for hh in range(2)]

    dstf = jnp.pad(edge_index[1], (0, _EPAD - _E))
    bases = jnp.array([[0], [_QTR], [2 * _QTR], [3 * _QTR]], jnp.int32)
    rel = dstf[None, :] - bases
    dump = _QTR + (jnp.arange(_EPAD, dtype=jnp.int32) % 128)[None, :]
    routed = jnp.where((rel >= 0) & (rel < _QTR), rel, dump).astype(jnp.int32)
    dst_h = []
    for hh in range(2):
        r_h = routed[:, hh * _EH:(hh + 1) * _EH]
        pages = r_h.reshape(2, 2, 16, _SNG * 2, _SUB).reshape(64, _SNG * 2,
                                                              _SUB)
        dst_h.append(jnp.pad(pages, ((0, 0), (0, 16 - _SNG * 2), (0, 0))))

    t2 = W_e2.reshape(_DH * _DH, _DH).astype(jnp.bfloat16)
    bm = b_e2.reshape(_DH, _DH)
    wt = W_msg[:_DH]
    wb = W_msg[_DH:]
    zeros_stripe = jnp.zeros((_ZSTR, _DP), f32)

    sc_gather, sc_scatter = _sc_kernels()
    out, outb = _init_tc(x_pad, W_in, b_in)
    h = _dense_relu(ea_pad, W_e1, b_e1, out_dtype=jnp.bfloat16)
    h_h = [h[:_EH], h[_EH:]]
    for step in range(3):
        aggs = []
        for hh in range(2):
            xe = sc_gather(outb, src_h[hh])
            msg = _msg_tc(xe, h_h[hh], t2, bm, hh)
            parts = sc_scatter(msg, dst_h[hh], zeros_stripe)
            aggs.append(parts.reshape(_NPAD, _DP))
        out, outb = _update_tc(aggs[0], aggs[1], out, W_root, conv_bias,
                               wt, wb, b_msg, x_pad, last=(step == 2))
    return out[:_N]
